# baseline (speedup 1.0000x reference)
import functools

import jax
import jax.numpy as jnp
from jax import lax
from jax.experimental import pallas as pl
from jax.experimental.pallas import tpu as pltpu
from jax.experimental.pallas import tpu_sc as plsc

NC = 2
NS = 16
NW = NC * NS
EB = 128


def _sigmoid(v):
    return jax.nn.sigmoid(v)


def _make_deg_kernel(n_pad, n_rows2d, nblk):
    rps = n_pad // NS

    mesh = plsc.VectorSubcoreMesh(
        core_axis_name="c", subcore_axis_name="s", num_cores=NC,
        num_subcores=NS)

    @functools.partial(
        pl.kernel, mesh=mesh,
        out_type=jax.ShapeDtypeStruct((NC, n_pad, 16), jnp.float32),
        scratch_types=[
            pltpu.MemorySpace.VMEM_SHARED((n_pad, 16), jnp.float32),
            pltpu.VMEM((nblk, EB), jnp.int32),
            pltpu.VMEM((EB, 16), jnp.float32),
        ],
    )
    def deg_kernel(dst2d_hbm, zeros16_hbm, ones16_hbm, out_hbm,
                   acc, idst, ones_v):
        c = lax.axis_index("c")
        s = lax.axis_index("s")
        wid = c * NS + s
        row0 = s * rps
        pltpu.sync_copy(dst2d_hbm.at[pl.ds(wid * nblk, nblk)], idst)
        pltpu.sync_copy(ones16_hbm, ones_v)
        pltpu.sync_copy(zeros16_hbm.at[pl.ds(row0, rps)],
                        acc.at[pl.ds(row0, rps)])
        plsc.subcore_barrier()

        def body(blk, _):
            pltpu.sync_copy(ones_v, acc.at[idst.at[blk]], add=True)
            return _

        lax.fori_loop(0, nblk, body, None)
        plsc.subcore_barrier()
        pltpu.sync_copy(acc.at[pl.ds(row0, rps)],
                        out_hbm.at[c, pl.ds(row0, rps)])

    return deg_kernel


def _make_prop_kernel(n_pad, n_rows2d, nblk, tsteps):
    rps = n_pad // NS

    mesh = plsc.VectorSubcoreMesh(
        core_axis_name="c", subcore_axis_name="s", num_cores=NC,
        num_subcores=NS)

    @functools.partial(
        pl.kernel, mesh=mesh,
        out_type=jax.ShapeDtypeStruct((tsteps, NC, n_pad, 128), jnp.float32),
        scratch_types=[
            pltpu.MemorySpace.VMEM_SHARED((n_pad, 128), jnp.float32),
            pltpu.VMEM((2, 1, EB), jnp.int32),
            pltpu.VMEM((2, 1, EB), jnp.int32),
            pltpu.VMEM((EB, 128), jnp.float32),
            pltpu.VMEM((EB, 128), jnp.float32),
            pltpu.SemaphoreType.DMA,
            pltpu.SemaphoreType.DMA,
            pltpu.SemaphoreType.DMA,
            pltpu.SemaphoreType.DMA,
        ],
    )
    def prop_kernel(src3d_hbm, dst2d_hbm, table_hbm, zeros_hbm, out_hbm,
                    acc, isrc, idst, rows_a, rows_b,
                    sem_ia, sem_ib, sem_a, sem_b):
        c = lax.axis_index("c")
        s = lax.axis_index("s")
        wid = c * NS + s
        row0 = s * rps
        blk0 = wid * nblk
        isem = [sem_ia, sem_ib]
        rows = [rows_a, rows_b]
        rsem = [sem_a, sem_b]

        for t in range(tsteps):
            def startidx(blk, par, _t=t):
                pltpu.async_copy(
                    src3d_hbm.at[_t, pl.ds(blk0 + blk, 1)],
                    isrc.at[par], isem[par])
                pltpu.async_copy(
                    dst2d_hbm.at[pl.ds(blk0 + blk, 1)],
                    idst.at[par], isem[par])

            def drainidx(par, _t=t):
                pltpu.make_async_copy(
                    src3d_hbm.at[_t, pl.ds(blk0, 1)],
                    isrc.at[par], isem[par]).wait()
                pltpu.make_async_copy(
                    dst2d_hbm.at[pl.ds(blk0, 1)],
                    idst.at[par], isem[par]).wait()

            def startrows(par):
                pltpu.async_copy(
                    table_hbm.at[isrc.at[par, 0]], rows[par], rsem[par])

            def drainrows(par):
                pltpu.make_async_copy(
                    table_hbm.at[isrc.at[par, 0]], rows[par],
                    rsem[par]).wait()

            def scatter(par):
                pltpu.sync_copy(rows[par], acc.at[idst.at[par, 0]], add=True)

            pltpu.sync_copy(zeros_hbm.at[pl.ds(row0, rps)],
                            acc.at[pl.ds(row0, rps)])
            plsc.subcore_barrier()

            startidx(0, 0)
            startidx(1, 1)
            drainidx(0)
            startrows(0)

            def body(k, _):
                drainidx(1)
                startrows(1)
                drainrows(0)
                scatter(0)
                startidx(2 * k + 2, 0)
                drainidx(0)
                startrows(0)
                drainrows(1)
                scatter(1)
                startidx(2 * k + 3, 1)
                return _

            lax.fori_loop(0, (nblk - 2) // 2, body, None)
            drainidx(1)
            startrows(1)
            drainrows(0)
            scatter(0)
            drainrows(1)
            scatter(1)
            plsc.subcore_barrier()
            pltpu.sync_copy(acc.at[pl.ds(row0, rps)],
                            out_hbm.at[t, c, pl.ds(row0, rps)])
            plsc.subcore_barrier()

    return prop_kernel


def _dinv_from_deg(dref):
    d = dref[...]
    return lax.rsqrt(1.0 + d[0, :, 0:1] + d[1, :, 0:1])


def _prep_body(xref, dref, gref):
    dinv = _dinv_from_deg(dref)
    gref[...] = (dinv * xref[0])[None]


def _step0_body(sxref, xref, dref, wref, bref, hout, gout):
    dinv = _dinv_from_deg(dref)
    d2 = dinv * dinv
    sx = sxref[0, 0] + sxref[0, 1]
    xp = dinv * sx + d2 * xref[0]
    w = wref[...]
    b = bref[...]
    br = b[0:1] + b[1:2]
    bz = b[2:3] + b[3:4]
    bn = b[4:5]
    r = _sigmoid(jnp.dot(xp, w[0], preferred_element_type=jnp.float32) + br)
    z = _sigmoid(jnp.dot(xp, w[2], preferred_element_type=jnp.float32) + bz)
    nn = jnp.tanh(jnp.dot(xp, w[4], preferred_element_type=jnp.float32)
                  + bn + r * bn)
    hn = z * nn
    hout[...] = hn
    gout[...] = dinv * hn


def _gates(sxref, xref, shref, href, dref, wref, bref):
    dinv = _dinv_from_deg(dref)
    d2 = dinv * dinv
    sx = sxref[0, 0] + sxref[0, 1]
    xp = dinv * sx + d2 * xref[0]
    hv = href[...]
    sh = shref[0] + shref[1]
    hp = dinv * sh + d2 * hv
    w = wref[...]
    b = bref[...]
    br = b[0:1] + b[1:2]
    bz = b[2:3] + b[3:4]
    bn = b[4:5]
    r = _sigmoid(jnp.dot(xp, w[0], preferred_element_type=jnp.float32)
                 + jnp.dot(hp, w[1], preferred_element_type=jnp.float32) + br)
    z = _sigmoid(jnp.dot(xp, w[2], preferred_element_type=jnp.float32)
                 + jnp.dot(hp, w[3], preferred_element_type=jnp.float32) + bz)
    nn = jnp.tanh(jnp.dot(xp, w[4], preferred_element_type=jnp.float32) + bn
                  + r * (jnp.dot(hp, w[4], preferred_element_type=jnp.float32)
                         + bn))
    hn = (1.0 - z) * hv + z * nn
    return hn, dinv


def _stepmid_body(sxref, xref, shref, href, dref, wref, bref, hout, gout):
    hn, dinv = _gates(sxref, xref, shref, href, dref, wref, bref)
    hout[...] = hn
    gout[...] = dinv * hn


def _steplast_body(sxref, xref, shref, href, dref, wref, bref,
                   wfcref, bfcref, yout):
    hn, _ = _gates(sxref, xref, shref, href, dref, wref, bref)
    yout[...] = (jnp.dot(hn, wfcref[...], preferred_element_type=jnp.float32)
                 + bfcref[...])


def kernel(x, edge_index, W_xr, b_xr, W_hr, b_hr, W_xz, b_xz, W_hz, b_hz,
           W_hn, b_hn, W_fc, b_fc):
    n, tsteps, in_dim = x.shape
    hid = W_hr.shape[0]
    e = edge_index.shape[1]

    n_pad = ((n + 1 + NS * 8 - 1) // (NS * 8)) * (NS * 8)
    nblk = -(-e // (NW * EB))
    if nblk % 2:
        nblk += 1
    e_pad = NW * nblk * EB
    n_rows2d = e_pad // EB

    src = edge_index[0].astype(jnp.int32)
    dst = edge_index[1].astype(jnp.int32)
    pad = e_pad - e
    srcp = jnp.concatenate([src, jnp.zeros((pad,), jnp.int32)])
    dstp = jnp.concatenate([dst, jnp.full((pad,), n, jnp.int32)])
    dst2d = dstp.reshape(n_rows2d, EB)
    srcx3d = (srcp[None, :]
              + (jnp.arange(tsteps, dtype=jnp.int32) * n)[:, None]
              ).reshape(tsteps, n_rows2d, EB)
    srch3d = srcp.reshape(1, n_rows2d, EB)

    zeros = jnp.zeros((n_pad, 128), jnp.float32)
    zeros16 = jnp.zeros((n_pad, 16), jnp.float32)
    ones16 = jnp.ones((EB, 16), jnp.float32)
    xT = jnp.transpose(x, (1, 0, 2))
    Wst = jnp.stack([W_xr, W_hr, W_xz, W_hz, W_hn])
    bst = jnp.stack([b_xr, b_hr, b_xz, b_hz, b_hn])
    bfc2 = b_fc.reshape(1, 128)

    degp = _make_deg_kernel(n_pad, n_rows2d, nblk)(dst2d, zeros16, ones16)

    BN = 1000
    nb = n // BN
    gx = pl.pallas_call(
        _prep_body,
        grid=(tsteps, nb),
        in_specs=[
            pl.BlockSpec((1, BN, 128), lambda t, i: (t, i, 0)),
            pl.BlockSpec((NC, BN, 16), lambda t, i: (0, i, 0)),
        ],
        out_specs=pl.BlockSpec((1, BN, 128), lambda t, i: (t, i, 0)),
        out_shape=jax.ShapeDtypeStruct((tsteps, n, 128), jnp.float32),
    )(xT, degp)
    gxf = gx.reshape(tsteps * n, 128)

    xprop = _make_prop_kernel(n_pad, n_rows2d, nblk, tsteps)
    sx = xprop(srcx3d, dst2d, gxf, zeros)

    hprop = _make_prop_kernel(n_pad, n_rows2d, nblk, 1)

    def step_specs(t, with_h):
        specs = [
            pl.BlockSpec((1, NC, BN, 128), lambda i, _t=t: (_t, 0, i, 0)),
            pl.BlockSpec((1, BN, 128), lambda i, _t=t: (_t, i, 0)),
        ]
        if with_h:
            specs += [
                pl.BlockSpec((NC, BN, 128), lambda i: (0, i, 0)),
                pl.BlockSpec((BN, 128), lambda i: (i, 0)),
            ]
        specs += [
            pl.BlockSpec((NC, BN, 16), lambda i: (0, i, 0)),
            pl.BlockSpec((5, 128, 128), lambda i: (0, 0, 0)),
            pl.BlockSpec((5, 128), lambda i: (0, 0)),
        ]
        return specs

    hg_out = (
        [pl.BlockSpec((BN, 128), lambda i: (i, 0))] * 2,
        [jax.ShapeDtypeStruct((n, 128), jnp.float32)] * 2,
    )

    h, g = pl.pallas_call(
        _step0_body,
        grid=(nb,),
        in_specs=step_specs(0, False),
        out_specs=hg_out[0],
        out_shape=hg_out[1],
    )(sx, xT, degp, Wst, bst)

    for t in range(1, tsteps - 1):
        sh = hprop(srch3d, dst2d, g, zeros)[0]
        h, g = pl.pallas_call(
            _stepmid_body,
            grid=(nb,),
            in_specs=step_specs(t, True),
            out_specs=hg_out[0],
            out_shape=hg_out[1],
        )(sx, xT, sh, h, degp, Wst, bst)

    sh = hprop(srch3d, dst2d, g, zeros)[0]
    y = pl.pallas_call(
        _steplast_body,
        grid=(nb,),
        in_specs=step_specs(tsteps - 1, True) + [
            pl.BlockSpec((128, 128), lambda i: (0, 0)),
            pl.BlockSpec((1, 128), lambda i: (0, 0)),
        ],
        out_specs=pl.BlockSpec((BN, 128), lambda i: (i, 0)),
        out_shape=jax.ShapeDtypeStruct((n, 128), jnp.float32),
    )(sx, xT, sh, h, degp, Wst, bst, W_fc, bfc2)

    half = n // 2
    mask0 = dst < half
    cs0 = jnp.cumsum(mask0.astype(jnp.int32))
    pos0 = jnp.where(mask0, cs0 - 1, e_pad)
    a0s = jnp.zeros((e_pad,), jnp.int32).at[pos0].set(src, mode='drop')
    a0d = jnp.full((e_pad,), n, jnp.int32).at[pos0].set(dst, mode='drop')
    cs1 = jnp.cumsum(1 - mask0.astype(jnp.int32))
    pos1 = jnp.where(mask0, e_pad, cs1 - 1)
    a1s = jnp.zeros((e_pad,), jnp.int32).at[pos1].set(src, mode='drop')
    a1d = jnp.full((e_pad,), n, jnp.int32).at[pos1].set(dst - half,
                                                       mode='drop')
    probe = (jnp.sum(a0s) + jnp.sum(a0d) + jnp.sum(a1s) + jnp.sum(a1d)
             ).astype(jnp.float32) * 0.0
    return y + probe


# speedup vs baseline: 3.5921x; 3.5921x over previous
import functools

import jax
import jax.numpy as jnp
from jax import lax
from jax.experimental import pallas as pl
from jax.experimental.pallas import tpu as pltpu
from jax.experimental.pallas import tpu_sc as plsc

NC = 2
NS = 16
NW = NC * NS
EB = 128


def _sigmoid(v):
    return jax.nn.sigmoid(v)


def _make_deg_kernel(n_pad, n_rows2d, nblk):
    rps = n_pad // NS

    mesh = plsc.VectorSubcoreMesh(
        core_axis_name="c", subcore_axis_name="s", num_cores=NC,
        num_subcores=NS)

    @functools.partial(
        pl.kernel, mesh=mesh,
        out_type=jax.ShapeDtypeStruct((NC, n_pad, 16), jnp.float32),
        scratch_types=[
            pltpu.MemorySpace.VMEM_SHARED((n_pad, 16), jnp.float32),
            pltpu.VMEM((nblk, EB), jnp.int32),
            pltpu.VMEM((EB, 16), jnp.float32),
        ],
    )
    def deg_kernel(dst2d_hbm, zeros16_hbm, ones16_hbm, out_hbm,
                   acc, idst, ones_v):
        c = lax.axis_index("c")
        s = lax.axis_index("s")
        wid = c * NS + s
        row0 = s * rps
        pltpu.sync_copy(dst2d_hbm.at[pl.ds(wid * nblk, nblk)], idst)
        pltpu.sync_copy(ones16_hbm, ones_v)
        pltpu.sync_copy(zeros16_hbm.at[pl.ds(row0, rps)],
                        acc.at[pl.ds(row0, rps)])
        plsc.subcore_barrier()

        def body(blk, _):
            pltpu.sync_copy(ones_v, acc.at[idst.at[blk]], add=True)
            return _

        lax.fori_loop(0, nblk, body, None)
        plsc.subcore_barrier()
        pltpu.sync_copy(acc.at[pl.ds(row0, rps)],
                        out_hbm.at[c, pl.ds(row0, rps)])

    return deg_kernel


def _make_prop_kernel(n_pad, n_rows2d, nb_pair, tsteps):
    rps = n_pad // NS
    nb0, nb1 = nb_pair

    mesh = plsc.VectorSubcoreMesh(
        core_axis_name="c", subcore_axis_name="s", num_cores=NC,
        num_subcores=NS)

    @functools.partial(
        pl.kernel, mesh=mesh,
        out_type=jax.ShapeDtypeStruct((tsteps, NC, n_pad, 128), jnp.float32),
        scratch_types=[
            pltpu.MemorySpace.VMEM_SHARED((n_pad, 128), jnp.float32),
            pltpu.VMEM((2, 1, EB), jnp.int32),
            pltpu.VMEM((2, 1, EB), jnp.int32),
            pltpu.VMEM((EB, 128), jnp.float32),
            pltpu.VMEM((EB, 128), jnp.float32),
            pltpu.SemaphoreType.DMA,
            pltpu.SemaphoreType.DMA,
            pltpu.SemaphoreType.DMA,
            pltpu.SemaphoreType.DMA,
        ],
    )
    def prop_kernel(src3d_hbm, dst2d_hbm, table_hbm, zeros_hbm, out_hbm,
                    acc, isrc, idst, rows_a, rows_b,
                    sem_ia, sem_ib, sem_a, sem_b):
        c = lax.axis_index("c")
        s = lax.axis_index("s")
        row0 = s * rps
        nblk = jnp.where(c == 0, nb0, nb1)
        blk0 = jnp.where(c == 0, s * nb0, NS * nb0 + s * nb1)
        isem = [sem_ia, sem_ib]
        rows = [rows_a, rows_b]
        rsem = [sem_a, sem_b]

        for t in range(tsteps):
            def startidx(blk, par, _t=t):
                pltpu.async_copy(
                    src3d_hbm.at[_t, pl.ds(blk0 + blk, 1)],
                    isrc.at[par], isem[par])
                pltpu.async_copy(
                    dst2d_hbm.at[pl.ds(blk0 + blk, 1)],
                    idst.at[par], isem[par])

            def drainidx(par, _t=t):
                pltpu.make_async_copy(
                    src3d_hbm.at[_t, pl.ds(blk0, 1)],
                    isrc.at[par], isem[par]).wait()
                pltpu.make_async_copy(
                    dst2d_hbm.at[pl.ds(blk0, 1)],
                    idst.at[par], isem[par]).wait()

            def startrows(par):
                pltpu.async_copy(
                    table_hbm.at[isrc.at[par, 0]], rows[par], rsem[par])

            def drainrows(par):
                pltpu.make_async_copy(
                    table_hbm.at[isrc.at[par, 0]], rows[par],
                    rsem[par]).wait()

            def scatter(par):
                pltpu.sync_copy(rows[par], acc.at[idst.at[par, 0]], add=True)

            pltpu.sync_copy(zeros_hbm.at[pl.ds(row0, rps)],
                            acc.at[pl.ds(row0, rps)])
            plsc.subcore_barrier()

            startidx(0, 0)
            startidx(1, 1)
            drainidx(0)
            startrows(0)

            def body(k, _):
                drainidx(1)
                startrows(1)
                drainrows(0)
                scatter(0)
                startidx(2 * k + 2, 0)
                drainidx(0)
                startrows(0)
                drainrows(1)
                scatter(1)
                startidx(2 * k + 3, 1)
                return _

            lax.fori_loop(0, (nblk - 2) // 2, body, None)
            drainidx(1)
            startrows(1)
            drainrows(0)
            scatter(0)
            drainrows(1)
            scatter(1)
            plsc.subcore_barrier()
            pltpu.sync_copy(acc.at[pl.ds(row0, rps)],
                            out_hbm.at[t, c, pl.ds(row0, rps)])
            plsc.subcore_barrier()

    return prop_kernel


def _dinv_from_deg(dref):
    d = dref[...]
    return lax.rsqrt(1.0 + d[0, :, 0:1] + d[1, :, 0:1])


def _prep_body(xref, dref, gref):
    dinv = _dinv_from_deg(dref)
    gref[...] = (dinv * xref[0])[None]


def _step0_body(sxref, xref, dref, wref, bref, hout, gout):
    dinv = _dinv_from_deg(dref)
    d2 = dinv * dinv
    sx = sxref[0, 0] + sxref[0, 1]
    xp = dinv * sx + d2 * xref[0]
    w = wref[...]
    b = bref[...]
    br = b[0:1] + b[1:2]
    bz = b[2:3] + b[3:4]
    bn = b[4:5]
    r = _sigmoid(jnp.dot(xp, w[0], preferred_element_type=jnp.float32) + br)
    z = _sigmoid(jnp.dot(xp, w[2], preferred_element_type=jnp.float32) + bz)
    nn = jnp.tanh(jnp.dot(xp, w[4], preferred_element_type=jnp.float32)
                  + bn + r * bn)
    hn = z * nn
    hout[...] = hn
    gout[...] = dinv * hn


def _gates(sxref, xref, shref, href, dref, wref, bref):
    dinv = _dinv_from_deg(dref)
    d2 = dinv * dinv
    sx = sxref[0, 0] + sxref[0, 1]
    xp = dinv * sx + d2 * xref[0]
    hv = href[...]
    sh = shref[0] + shref[1]
    hp = dinv * sh + d2 * hv
    w = wref[...]
    b = bref[...]
    br = b[0:1] + b[1:2]
    bz = b[2:3] + b[3:4]
    bn = b[4:5]
    r = _sigmoid(jnp.dot(xp, w[0], preferred_element_type=jnp.float32)
                 + jnp.dot(hp, w[1], preferred_element_type=jnp.float32) + br)
    z = _sigmoid(jnp.dot(xp, w[2], preferred_element_type=jnp.float32)
                 + jnp.dot(hp, w[3], preferred_element_type=jnp.float32) + bz)
    nn = jnp.tanh(jnp.dot(xp, w[4], preferred_element_type=jnp.float32) + bn
                  + r * (jnp.dot(hp, w[4], preferred_element_type=jnp.float32)
                         + bn))
    hn = (1.0 - z) * hv + z * nn
    return hn, dinv


def _stepmid_body(sxref, xref, shref, href, dref, wref, bref, hout, gout):
    hn, dinv = _gates(sxref, xref, shref, href, dref, wref, bref)
    hout[...] = hn
    gout[...] = dinv * hn


def _steplast_body(sxref, xref, shref, href, dref, wref, bref,
                   wfcref, bfcref, yout):
    hn, _ = _gates(sxref, xref, shref, href, dref, wref, bref)
    yout[...] = (jnp.dot(hn, wfcref[...], preferred_element_type=jnp.float32)
                 + bfcref[...])


def kernel(x, edge_index, W_xr, b_xr, W_hr, b_hr, W_xz, b_xz, W_hz, b_hz,
           W_hn, b_hn, W_fc, b_fc):
    n, tsteps, in_dim = x.shape
    hid = W_hr.shape[0]
    e = edge_index.shape[1]

    n_pad = ((n + 1 + NS * 8 - 1) // (NS * 8)) * (NS * 8)
    nblk = -(-e // (NW * EB))
    if nblk % 2:
        nblk += 1
    e_pad = NW * nblk * EB
    n_rows2d = e_pad // EB

    nb_total = 2 * nblk
    nb0 = (int(nb_total * 0.825) // 2) * 2
    nb1 = nb_total - nb0

    src = edge_index[0].astype(jnp.int32)
    dst = edge_index[1].astype(jnp.int32)
    pad = e_pad - e
    padr = jnp.arange(pad, dtype=jnp.int32)
    srcp = jnp.concatenate([src, padr % n])
    dstp = jnp.concatenate([dst, n + padr % (n_pad - n)])
    dst2d = dstp.reshape(n_rows2d, EB)
    srcx3d = (srcp[None, :]
              + (jnp.arange(tsteps, dtype=jnp.int32) * n)[:, None]
              ).reshape(tsteps, n_rows2d, EB)
    srch3d = srcp.reshape(1, n_rows2d, EB)

    zeros = jnp.zeros((n_pad, 128), jnp.float32)
    zeros16 = jnp.zeros((n_pad, 16), jnp.float32)
    ones16 = jnp.ones((EB, 16), jnp.float32)
    xT = jnp.transpose(x, (1, 0, 2))
    Wst = jnp.stack([W_xr, W_hr, W_xz, W_hz, W_hn])
    bst = jnp.stack([b_xr, b_hr, b_xz, b_hz, b_hn])
    bfc2 = b_fc.reshape(1, 128)

    degp = _make_deg_kernel(n_pad, n_rows2d, nblk)(dst2d, zeros16, ones16)

    BN = 1000
    nb = n // BN
    gx = pl.pallas_call(
        _prep_body,
        grid=(tsteps, nb),
        in_specs=[
            pl.BlockSpec((1, BN, 128), lambda t, i: (t, i, 0)),
            pl.BlockSpec((NC, BN, 16), lambda t, i: (0, i, 0)),
        ],
        out_specs=pl.BlockSpec((1, BN, 128), lambda t, i: (t, i, 0)),
        out_shape=jax.ShapeDtypeStruct((tsteps, n, 128), jnp.float32),
    )(xT, degp)
    gxf = gx.reshape(tsteps * n, 128)

    xprop = _make_prop_kernel(n_pad, n_rows2d, (nb0, nb1), tsteps)
    sx = xprop(srcx3d, dst2d, gxf, zeros)

    hprop = _make_prop_kernel(n_pad, n_rows2d, (nb0, nb1), 1)

    def step_specs(t, with_h):
        specs = [
            pl.BlockSpec((1, NC, BN, 128), lambda i, _t=t: (_t, 0, i, 0)),
            pl.BlockSpec((1, BN, 128), lambda i, _t=t: (_t, i, 0)),
        ]
        if with_h:
            specs += [
                pl.BlockSpec((NC, BN, 128), lambda i: (0, i, 0)),
                pl.BlockSpec((BN, 128), lambda i: (i, 0)),
            ]
        specs += [
            pl.BlockSpec((NC, BN, 16), lambda i: (0, i, 0)),
            pl.BlockSpec((5, 128, 128), lambda i: (0, 0, 0)),
            pl.BlockSpec((5, 128), lambda i: (0, 0)),
        ]
        return specs

    hg_out = (
        [pl.BlockSpec((BN, 128), lambda i: (i, 0))] * 2,
        [jax.ShapeDtypeStruct((n, 128), jnp.float32)] * 2,
    )

    h, g = pl.pallas_call(
        _step0_body,
        grid=(nb,),
        in_specs=step_specs(0, False),
        out_specs=hg_out[0],
        out_shape=hg_out[1],
    )(sx, xT, degp, Wst, bst)

    for t in range(1, tsteps - 1):
        sh = hprop(srch3d, dst2d, g, zeros)[0]
        h, g = pl.pallas_call(
            _stepmid_body,
            grid=(nb,),
            in_specs=step_specs(t, True),
            out_specs=hg_out[0],
            out_shape=hg_out[1],
        )(sx, xT, sh, h, degp, Wst, bst)

    sh = hprop(srch3d, dst2d, g, zeros)[0]
    y = pl.pallas_call(
        _steplast_body,
        grid=(nb,),
        in_specs=step_specs(tsteps - 1, True) + [
            pl.BlockSpec((128, 128), lambda i: (0, 0)),
            pl.BlockSpec((1, 128), lambda i: (0, 0)),
        ],
        out_specs=pl.BlockSpec((BN, 128), lambda i: (i, 0)),
        out_shape=jax.ShapeDtypeStruct((n, 128), jnp.float32),
    )(sx, xT, sh, h, degp, Wst, bst, W_fc, bfc2)
    return y


# speedup vs baseline: 5.1013x; 1.4201x over previous
import functools

import jax
import jax.numpy as jnp
from jax import lax
from jax.experimental import pallas as pl
from jax.experimental.pallas import tpu as pltpu
from jax.experimental.pallas import tpu_sc as plsc

NC = 2
NS = 16
NW = NC * NS
EB = 128


def _sigmoid(v):
    return jax.nn.sigmoid(v)


def _make_deg_kernel(n_pad, n_rows2d, nblk):
    rps = n_pad // NS

    mesh = plsc.VectorSubcoreMesh(
        core_axis_name="c", subcore_axis_name="s", num_cores=NC,
        num_subcores=NS)

    @functools.partial(
        pl.kernel, mesh=mesh,
        out_type=jax.ShapeDtypeStruct((NC, n_pad, 16), jnp.float32),
        scratch_types=[
            pltpu.MemorySpace.VMEM_SHARED((n_pad, 16), jnp.float32),
            pltpu.VMEM((nblk, EB), jnp.int32),
            pltpu.VMEM((EB, 16), jnp.float32),
        ],
    )
    def deg_kernel(dst2d_hbm, zeros16_hbm, ones16_hbm, out_hbm,
                   acc, idst, ones_v):
        c = lax.axis_index("c")
        s = lax.axis_index("s")
        wid = c * NS + s
        row0 = s * rps
        pltpu.sync_copy(dst2d_hbm.at[pl.ds(wid * nblk, nblk)], idst)
        pltpu.sync_copy(ones16_hbm, ones_v)
        pltpu.sync_copy(zeros16_hbm.at[pl.ds(row0, rps)],
                        acc.at[pl.ds(row0, rps)])
        plsc.subcore_barrier()

        def body(blk, _):
            pltpu.sync_copy(ones_v, acc.at[idst.at[blk]], add=True)
            return _

        lax.fori_loop(0, nblk, body, None)
        plsc.subcore_barrier()
        pltpu.sync_copy(acc.at[pl.ds(row0, rps)],
                        out_hbm.at[c, pl.ds(row0, rps)])

    return deg_kernel


def _make_prop_kernel(n_pad, n_rows2d, nb_pair, tsteps):
    rps = n_pad // NS
    nb0, nb1 = nb_pair

    mesh = plsc.VectorSubcoreMesh(
        core_axis_name="c", subcore_axis_name="s", num_cores=NC,
        num_subcores=NS)

    @functools.partial(
        pl.kernel, mesh=mesh,
        out_type=jax.ShapeDtypeStruct((tsteps, NC, n_pad, 128), jnp.float32),
        scratch_types=[
            pltpu.MemorySpace.VMEM_SHARED((n_pad, 128), jnp.float32),
            pltpu.VMEM((2, 1, EB), jnp.int32),
            pltpu.VMEM((2, 1, EB), jnp.int32),
            pltpu.VMEM((EB, 128), jnp.float32),
            pltpu.VMEM((EB, 128), jnp.float32),
            pltpu.SemaphoreType.DMA,
            pltpu.SemaphoreType.DMA,
            pltpu.SemaphoreType.DMA,
            pltpu.SemaphoreType.DMA,
        ],
    )
    def prop_kernel(src3d_hbm, dst2d_hbm, table_hbm, zeros_hbm, out_hbm,
                    acc, isrc, idst, rows_a, rows_b,
                    sem_ia, sem_ib, sem_a, sem_b):
        c = lax.axis_index("c")
        s = lax.axis_index("s")
        row0 = s * rps
        nblk = jnp.where(c == 0, nb0, nb1)
        blk0 = jnp.where(c == 0, s * nb0, NS * nb0 + s * nb1)
        isem = [sem_ia, sem_ib]
        rows = [rows_a, rows_b]
        rsem = [sem_a, sem_b]

        for t in range(tsteps):
            def startidx(blk, par, _t=t):
                pltpu.async_copy(
                    src3d_hbm.at[_t, pl.ds(blk0 + blk, 1)],
                    isrc.at[par], isem[par])
                pltpu.async_copy(
                    dst2d_hbm.at[pl.ds(blk0 + blk, 1)],
                    idst.at[par], isem[par])

            def drainidx(par, _t=t):
                pltpu.make_async_copy(
                    src3d_hbm.at[_t, pl.ds(blk0, 1)],
                    isrc.at[par], isem[par]).wait()
                pltpu.make_async_copy(
                    dst2d_hbm.at[pl.ds(blk0, 1)],
                    idst.at[par], isem[par]).wait()

            def startrows(par):
                pltpu.async_copy(
                    table_hbm.at[isrc.at[par, 0]], rows[par], rsem[par])

            def drainrows(par):
                pltpu.make_async_copy(
                    table_hbm.at[isrc.at[par, 0]], rows[par],
                    rsem[par]).wait()

            def scatter(par):
                pltpu.sync_copy(rows[par], acc.at[idst.at[par, 0]], add=True)

            pltpu.sync_copy(zeros_hbm.at[pl.ds(row0, rps)],
                            acc.at[pl.ds(row0, rps)])
            plsc.subcore_barrier()

            startidx(0, 0)
            startidx(1, 1)
            drainidx(0)
            startrows(0)

            def body(k, _):
                drainidx(1)
                startrows(1)
                drainrows(0)
                scatter(0)
                startidx(2 * k + 2, 0)
                drainidx(0)
                startrows(0)
                drainrows(1)
                scatter(1)
                startidx(2 * k + 3, 1)
                return _

            lax.fori_loop(0, (nblk - 2) // 2, body, None)
            drainidx(1)
            startrows(1)
            drainrows(0)
            scatter(0)
            drainrows(1)
            scatter(1)
            plsc.subcore_barrier()
            pltpu.sync_copy(acc.at[pl.ds(row0, rps)],
                            out_hbm.at[t, c, pl.ds(row0, rps)])
            plsc.subcore_barrier()

    return prop_kernel


def _dinv_from_deg(dref):
    d = dref[...]
    return lax.rsqrt(1.0 + d[0, :, 0:1] + d[1, :, 0:1])


def _prep_body(xref, dref, gref):
    dinv = _dinv_from_deg(dref)
    gref[...] = (dinv * xref[0])[None]


def _step0_body(sxref, xref, dref, wref, bref, hout, gout):
    dinv = _dinv_from_deg(dref)
    d2 = dinv * dinv
    sx = sxref[0, 0] + sxref[0, 1]
    xp = dinv * sx + d2 * xref[0]
    w = wref[...]
    b = bref[...]
    br = b[0:1] + b[1:2]
    bz = b[2:3] + b[3:4]
    bn = b[4:5]
    r = _sigmoid(jnp.dot(xp, w[0], preferred_element_type=jnp.float32) + br)
    z = _sigmoid(jnp.dot(xp, w[2], preferred_element_type=jnp.float32) + bz)
    nn = jnp.tanh(jnp.dot(xp, w[4], preferred_element_type=jnp.float32)
                  + bn + r * bn)
    hn = z * nn
    hout[...] = hn
    gout[...] = dinv * hn


def _gates(sxref, xref, shref, href, dref, wref, bref):
    dinv = _dinv_from_deg(dref)
    d2 = dinv * dinv
    sx = sxref[0, 0] + sxref[0, 1]
    xp = dinv * sx + d2 * xref[0]
    hv = href[...]
    sh = shref[0] + shref[1]
    hp = dinv * sh + d2 * hv
    w = wref[...]
    b = bref[...]
    br = b[0:1] + b[1:2]
    bz = b[2:3] + b[3:4]
    bn = b[4:5]
    r = _sigmoid(jnp.dot(xp, w[0], preferred_element_type=jnp.float32)
                 + jnp.dot(hp, w[1], preferred_element_type=jnp.float32) + br)
    z = _sigmoid(jnp.dot(xp, w[2], preferred_element_type=jnp.float32)
                 + jnp.dot(hp, w[3], preferred_element_type=jnp.float32) + bz)
    nn = jnp.tanh(jnp.dot(xp, w[4], preferred_element_type=jnp.float32) + bn
                  + r * (jnp.dot(hp, w[4], preferred_element_type=jnp.float32)
                         + bn))
    hn = (1.0 - z) * hv + z * nn
    return hn, dinv


def _stepmid_body(sxref, xref, shref, href, dref, wref, bref, hout, gout):
    hn, dinv = _gates(sxref, xref, shref, href, dref, wref, bref)
    hout[...] = hn
    gout[...] = dinv * hn


def _steplast_body(sxref, xref, shref, href, dref, wref, bref,
                   wfcref, bfcref, yout):
    hn, _ = _gates(sxref, xref, shref, href, dref, wref, bref)
    yout[...] = (jnp.dot(hn, wfcref[...], preferred_element_type=jnp.float32)
                 + bfcref[...])


def kernel(x, edge_index, W_xr, b_xr, W_hr, b_hr, W_xz, b_xz, W_hz, b_hz,
           W_hn, b_hn, W_fc, b_fc):
    n, tsteps, in_dim = x.shape
    hid = W_hr.shape[0]
    e = edge_index.shape[1]

    n_pad = ((n + 1 + NS * 8 - 1) // (NS * 8)) * (NS * 8)
    nblk = -(-e // (NW * EB))
    if nblk % 2:
        nblk += 1
    e_pad = NW * nblk * EB
    n_rows2d = e_pad // EB

    nb0 = nb1 = nblk

    src = edge_index[0].astype(jnp.int32)
    dst = edge_index[1].astype(jnp.int32)
    pad = e_pad - e
    padr = jnp.arange(pad, dtype=jnp.int32)
    srcp = jnp.concatenate([src, padr % n])
    dstp = jnp.concatenate([dst, n + padr % (n_pad - n)])
    dst2d = dstp.reshape(n_rows2d, EB)
    srcx3d = (srcp[None, :]
              + (jnp.arange(tsteps, dtype=jnp.int32) * n)[:, None]
              ).reshape(tsteps, n_rows2d, EB)
    srch3d = srcp.reshape(1, n_rows2d, EB)

    zeros = jnp.zeros((n_pad, 128), jnp.float32)
    zeros16 = jnp.zeros((n_pad, 16), jnp.float32)
    ones16 = jnp.ones((EB, 16), jnp.float32)
    xT = jnp.transpose(x, (1, 0, 2))
    Wst = jnp.stack([W_xr, W_hr, W_xz, W_hz, W_hn])
    bst = jnp.stack([b_xr, b_hr, b_xz, b_hz, b_hn])
    bfc2 = b_fc.reshape(1, 128)

    degp = _make_deg_kernel(n_pad, n_rows2d, nblk)(dst2d, zeros16, ones16)

    BN = 1000
    nb = n // BN
    gx = pl.pallas_call(
        _prep_body,
        grid=(tsteps, nb),
        in_specs=[
            pl.BlockSpec((1, BN, 128), lambda t, i: (t, i, 0)),
            pl.BlockSpec((NC, BN, 16), lambda t, i: (0, i, 0)),
        ],
        out_specs=pl.BlockSpec((1, BN, 128), lambda t, i: (t, i, 0)),
        out_shape=jax.ShapeDtypeStruct((tsteps, n, 128), jnp.float32),
    )(xT, degp)
    gxf = gx.reshape(tsteps * n, 128)

    xprop = _make_prop_kernel(n_pad, n_rows2d, (nb0, nb1), tsteps)
    sx = xprop(srcx3d, dst2d, gxf, zeros)

    hprop = _make_prop_kernel(n_pad, n_rows2d, (nb0, nb1), 1)

    def step_specs(t, with_h):
        specs = [
            pl.BlockSpec((1, NC, BN, 128), lambda i, _t=t: (_t, 0, i, 0)),
            pl.BlockSpec((1, BN, 128), lambda i, _t=t: (_t, i, 0)),
        ]
        if with_h:
            specs += [
                pl.BlockSpec((NC, BN, 128), lambda i: (0, i, 0)),
                pl.BlockSpec((BN, 128), lambda i: (i, 0)),
            ]
        specs += [
            pl.BlockSpec((NC, BN, 16), lambda i: (0, i, 0)),
            pl.BlockSpec((5, 128, 128), lambda i: (0, 0, 0)),
            pl.BlockSpec((5, 128), lambda i: (0, 0)),
        ]
        return specs

    hg_out = (
        [pl.BlockSpec((BN, 128), lambda i: (i, 0))] * 2,
        [jax.ShapeDtypeStruct((n, 128), jnp.float32)] * 2,
    )

    h, g = pl.pallas_call(
        _step0_body,
        grid=(nb,),
        in_specs=step_specs(0, False),
        out_specs=hg_out[0],
        out_shape=hg_out[1],
    )(sx, xT, degp, Wst, bst)

    for t in range(1, tsteps - 1):
        sh = hprop(srch3d, dst2d, g, zeros)[0]
        h, g = pl.pallas_call(
            _stepmid_body,
            grid=(nb,),
            in_specs=step_specs(t, True),
            out_specs=hg_out[0],
            out_shape=hg_out[1],
        )(sx, xT, sh, h, degp, Wst, bst)

    sh = hprop(srch3d, dst2d, g, zeros)[0]
    y = pl.pallas_call(
        _steplast_body,
        grid=(nb,),
        in_specs=step_specs(tsteps - 1, True) + [
            pl.BlockSpec((128, 128), lambda i: (0, 0)),
            pl.BlockSpec((1, 128), lambda i: (0, 0)),
        ],
        out_specs=pl.BlockSpec((BN, 128), lambda i: (i, 0)),
        out_shape=jax.ShapeDtypeStruct((n, 128), jnp.float32),
    )(sx, xT, sh, h, degp, Wst, bst, W_fc, bfc2)
    return y
